# Initial kernel scaffold; baseline (speedup 1.0000x reference)
import jax
import jax.numpy as jnp
from jax.experimental import pallas as pl


def kernel(x, edge_index, W, b, gamma, beta):
    raise NotImplementedError("write your pallas kernel here")


# speedup vs baseline: 12.9347x; 12.9347x over previous
import functools

import jax
import jax.numpy as jnp
from jax import lax
from jax.experimental import pallas as pl
from jax.experimental.pallas import tpu as pltpu
from jax.experimental.pallas import tpu_sc as plsc

N = 10000
E = 320000
D = 128

K = 128
NW = 32
EPW = 10240
CH = EPW // K
PH = 2
CHP = CH // PH
E_PAD = NW * EPW
NROW = 10112
RPT = NROW // 16
BLK = 1000
GRID = N // BLK


def _deg_body(dst_hbm, out_hbm, didx_v, hist_v):
    c = lax.axis_index("c")
    s = lax.axis_index("s")
    wid = s * 2 + c

    def zbody(i, carry):
        hist_v[pl.ds(i * 16, 16)] = jnp.zeros((16,), jnp.float32)
        return carry

    lax.fori_loop(0, (NROW + 16) // 16, zbody, 0)
    pltpu.sync_copy(dst_hbm.at[pl.ds(wid * EPW, EPW)], didx_v)
    e0 = jnp.where(lax.iota(jnp.int32, 16) == 0, 1.0, 0.0).astype(jnp.float32)

    def body(i, carry):
        ivec = didx_v[pl.ds(i * 16, 16)]
        for lane in range(16):
            jj = ivec[lane]
            hist_v[pl.ds(jj, 16)] = hist_v[pl.ds(jj, 16)] + e0
        return carry

    lax.fori_loop(0, EPW // 16, body, 0)
    pltpu.sync_copy(hist_v.at[pl.ds(0, NROW)], out_hbm.at[wid])


def _main_body(g_hbm, src_hbm, dst_hbm, zeros_hbm, out_hbm,
               sidx0, sidx1, didx0, didx1, buf0, buf1,
               sis0, sis1, sid0, sid1, sg0, sg1, acc_sh):
    c = lax.axis_index("c")
    s = lax.axis_index("s")
    wid = s * 2 + c
    base = s * RPT
    ebase = wid * EPW
    pltpu.sync_copy(zeros_hbm.at[pl.ds(base, RPT)], acc_sh.at[pl.ds(base, RPT)])
    plsc.subcore_barrier()

    sidx = (sidx0, sidx1)
    didx = (didx0, didx1)
    bufs = (buf0, buf1)
    sis = (sis0, sis1)
    sid = (sid0, sid1)
    sg = (sg0, sg1)

    def fire_idx(j, b):
        pltpu.async_copy(src_hbm.at[pl.ds(ebase + j * K, K)], sidx[b], sis[b])
        pltpu.async_copy(dst_hbm.at[pl.ds(ebase + j * K, K)], didx[b], sid[b])

    def wait_idx(b):
        pltpu.make_async_copy(src_hbm.at[pl.ds(0, K)], sidx[b], sis[b]).wait()
        pltpu.make_async_copy(dst_hbm.at[pl.ds(0, K)], didx[b], sid[b]).wait()

    def fire_gat(b):
        pltpu.async_copy(g_hbm.at[sidx[b]], bufs[b], sg[b])

    def wait_gat(b):
        pltpu.make_async_copy(g_hbm.at[sidx[b]], bufs[b], sg[b]).wait()

    def scat(b):
        pltpu.sync_copy(bufs[b], acc_sh.at[didx[b]], add=True)

    fire_idx(0, 0)
    wait_idx(0)
    fire_idx(1, 1)
    fire_gat(0)

    def body(m, carry):
        for b in range(2):
            wait_idx((b + 1) % 2)
            fire_gat((b + 1) % 2)
            wait_gat(b)
            scat(b)
            fire_idx(2 * m + b + 2, b)
        return carry

    lax.fori_loop(0, (CH - 2) // 2, body, 0)
    wait_idx(1)
    fire_gat(1)
    wait_gat(0)
    scat(0)
    wait_gat(1)
    scat(1)

    plsc.subcore_barrier()
    pltpu.sync_copy(acc_sh.at[pl.ds(base, RPT)], out_hbm.at[c, pl.ds(base, RPT)])


def _tc_g(x_ref, w_ref, hist_ref, g_ref, dis_ref):
    h = jnp.dot(x_ref[...], w_ref[...], preferred_element_type=jnp.float32)
    deg = jnp.sum(hist_ref[...], axis=1) + 1.0
    dis = lax.rsqrt(deg)[:, None]
    g_ref[...] = h * dis
    dis_ref[...] = jnp.broadcast_to(dis, h.shape)


def _tc_comb(p_ref, g_ref, dis_ref, b_ref, t_ref, s_ref, q_ref):
    t = dis_ref[...] * (p_ref[0] + p_ref[1] + g_ref[...]) + b_ref[...]
    t_ref[...] = t
    s_ref[...] = jnp.sum(t, axis=0).reshape(1, 1, D)
    q_ref[...] = jnp.sum(t * t, axis=0).reshape(1, 1, D)


def _tc_bn(t_ref, s_ref, q_ref, gam_ref, bet_ref, o_ref):
    inv_n = 1.0 / N
    mean = jnp.sum(s_ref[...], axis=0) * inv_n
    var = jnp.sum(q_ref[...], axis=0) * inv_n - mean * mean
    y = (t_ref[...] - mean) * lax.rsqrt(var + 1e-5) * gam_ref[...] + bet_ref[...]
    o_ref[...] = jnp.maximum(y, 0.0)


_mesh = plsc.VectorSubcoreMesh(core_axis_name="c", subcore_axis_name="s")

_deg_kernel = functools.partial(
    pl.kernel,
    out_type=jax.ShapeDtypeStruct((NW, NROW), jnp.float32),
    mesh=_mesh,
    scratch_types=[
        pltpu.VMEM((EPW,), jnp.int32),
        pltpu.VMEM((NROW + 16,), jnp.float32),
    ],
)(_deg_body)

_IDX = pltpu.VMEM((K,), jnp.int32)
_SEM = pltpu.SemaphoreType.DMA

_main_kernel = functools.partial(
    pl.kernel,
    out_type=jax.ShapeDtypeStruct((2, NROW, D), jnp.float32),
    mesh=_mesh,
    scratch_types=[
        _IDX, _IDX, _IDX, _IDX,
        pltpu.VMEM((K, D), jnp.float32),
        pltpu.VMEM((K, D), jnp.float32),
        _SEM, _SEM, _SEM, _SEM, _SEM, _SEM,
        pltpu.VMEM_SHARED((NROW, D), jnp.float32),
    ],
)(_main_body)


def kernel_debug(x, edge_index, W, b, gamma, beta, use_sc_deg, use_sc_main):
    import jax.ops
    src0, dst0 = edge_index[0], edge_index[1]
    srcp = jnp.concatenate([src0, jnp.zeros((E_PAD - E,), jnp.int32)])
    dstp = jnp.concatenate([dst0, jnp.full((E_PAD - E,), N, jnp.int32)])
    zeros16 = jnp.zeros((NROW, 16), jnp.float32)
    ones16 = jnp.ones((K, 16), jnp.float32)
    zeros128 = jnp.zeros((NROW, D), jnp.float32)

    if use_sc_deg:
        degp = _deg_kernel(dstp)
        deg = jnp.sum(degp[:, :N], axis=0) + 1.0
    else:
        ones = jnp.ones((E,), jnp.float32)
        deg = jax.ops.segment_sum(ones, dst0, num_segments=N) + 1.0
    dis = jax.lax.rsqrt(deg)
    g = (x @ W) * dis[:, None]
    if use_sc_main:
        parts = _main_kernel(g, srcp, dstp, zeros128)
        S = parts[0, :N] + parts[1, :N]
    else:
        S = jax.ops.segment_sum(g[src0], dst0, num_segments=N)
    out = dis[:, None] * (S + g) + b
    mean = jnp.mean(out, axis=0)
    var = jnp.var(out, axis=0)
    out = (out - mean) * jax.lax.rsqrt(var + 1e-5) * gamma + beta
    return jnp.maximum(out, 0.0)


def kernel(x, edge_index, W, b, gamma, beta):
    src = jnp.concatenate([edge_index[0], jnp.zeros((E_PAD - E,), jnp.int32)])
    dst = jnp.concatenate([edge_index[1], jnp.full((E_PAD - E,), N, jnp.int32)])

    zeros128 = jnp.zeros((NROW, D), jnp.float32)
    b2 = b.reshape(1, D)
    gamma2 = gamma.reshape(1, D)
    beta2 = beta.reshape(1, D)

    degp = _deg_kernel(dst).T

    g, dis = pl.pallas_call(
        _tc_g,
        grid=(GRID,),
        in_specs=[
            pl.BlockSpec((BLK, D), lambda i: (i, 0)),
            pl.BlockSpec((D, D), lambda i: (0, 0)),
            pl.BlockSpec((BLK, NW), lambda i: (i, 0)),
        ],
        out_specs=[
            pl.BlockSpec((BLK, D), lambda i: (i, 0)),
            pl.BlockSpec((BLK, D), lambda i: (i, 0)),
        ],
        out_shape=[
            jax.ShapeDtypeStruct((N, D), jnp.float32),
            jax.ShapeDtypeStruct((N, D), jnp.float32),
        ],
    )(x, W, degp)

    parts = _main_kernel(g, src, dst, zeros128)

    t, sums, sq = pl.pallas_call(
        _tc_comb,
        grid=(GRID,),
        in_specs=[
            pl.BlockSpec((2, BLK, D), lambda i: (0, i, 0)),
            pl.BlockSpec((BLK, D), lambda i: (i, 0)),
            pl.BlockSpec((BLK, D), lambda i: (i, 0)),
            pl.BlockSpec((1, D), lambda i: (0, 0)),
        ],
        out_specs=[
            pl.BlockSpec((BLK, D), lambda i: (i, 0)),
            pl.BlockSpec((1, 1, D), lambda i: (i, 0, 0)),
            pl.BlockSpec((1, 1, D), lambda i: (i, 0, 0)),
        ],
        out_shape=[
            jax.ShapeDtypeStruct((N, D), jnp.float32),
            jax.ShapeDtypeStruct((GRID, 1, D), jnp.float32),
            jax.ShapeDtypeStruct((GRID, 1, D), jnp.float32),
        ],
    )(parts, g, dis, b2)

    out = pl.pallas_call(
        _tc_bn,
        grid=(GRID,),
        in_specs=[
            pl.BlockSpec((BLK, D), lambda i: (i, 0)),
            pl.BlockSpec((GRID, 1, D), lambda i: (0, 0, 0)),
            pl.BlockSpec((GRID, 1, D), lambda i: (0, 0, 0)),
            pl.BlockSpec((1, D), lambda i: (0, 0)),
            pl.BlockSpec((1, D), lambda i: (0, 0)),
        ],
        out_specs=pl.BlockSpec((BLK, D), lambda i: (i, 0)),
        out_shape=jax.ShapeDtypeStruct((N, D), jnp.float32),
    )(t, sums, sq, gamma2, beta2)

    return out


# speedup vs baseline: 13.3719x; 1.0338x over previous
import functools

import jax
import jax.numpy as jnp
from jax import lax
from jax.experimental import pallas as pl
from jax.experimental.pallas import tpu as pltpu
from jax.experimental.pallas import tpu_sc as plsc

N = 10000
E = 320000
D = 128

K = 128
NW = 32
EPW = 10240
CH = EPW // K
PH = 2
CHP = CH // PH
E_PAD = NW * EPW
NROW = 10112
RPT = NROW // 16
BLK = 1000
GRID = N // BLK


def _deg_body(dst_hbm, out_hbm, didx_v, hist_a, hist_b):
    c = lax.axis_index("c")
    s = lax.axis_index("s")
    wid = s * 2 + c

    def zbody(i, carry):
        hist_a[pl.ds(i * 16, 16)] = jnp.zeros((16,), jnp.float32)
        hist_b[pl.ds(i * 16, 16)] = jnp.zeros((16,), jnp.float32)
        return carry

    lax.fori_loop(0, (NROW + 16) // 16, zbody, 0)
    pltpu.sync_copy(dst_hbm.at[pl.ds(wid * EPW, EPW)], didx_v)
    e0 = jnp.where(lax.iota(jnp.int32, 16) == 0, 1.0, 0.0).astype(jnp.float32)
    half = EPW // 2

    def body(i, carry):
        ivec_a = didx_v[pl.ds(i * 16, 16)]
        ivec_b = didx_v[pl.ds(half + i * 16, 16)]
        for lane in range(16):
            ja = ivec_a[lane]
            jb = ivec_b[lane]
            hist_a[pl.ds(ja, 16)] = hist_a[pl.ds(ja, 16)] + e0
            hist_b[pl.ds(jb, 16)] = hist_b[pl.ds(jb, 16)] + e0
        return carry

    lax.fori_loop(0, half // 16, body, 0)

    def mbody(i, carry):
        hist_a[pl.ds(i * 16, 16)] = (hist_a[pl.ds(i * 16, 16)]
                                     + hist_b[pl.ds(i * 16, 16)])
        return carry

    lax.fori_loop(0, NROW // 16, mbody, 0)
    pltpu.sync_copy(hist_a.at[pl.ds(0, NROW)], out_hbm.at[wid])


def _main_body(g_hbm, src_hbm, dst_hbm, zeros_hbm, out_hbm,
               sidx0, sidx1, didx0, didx1, buf0, buf1,
               sis0, sis1, sid0, sid1, sg0, sg1, acc_sh):
    c = lax.axis_index("c")
    s = lax.axis_index("s")
    wid = s * 2 + c
    base = s * RPT
    ebase = wid * EPW
    pltpu.sync_copy(zeros_hbm.at[pl.ds(base, RPT)], acc_sh.at[pl.ds(base, RPT)])
    plsc.subcore_barrier()

    sidx = (sidx0, sidx1)
    didx = (didx0, didx1)
    bufs = (buf0, buf1)
    sis = (sis0, sis1)
    sid = (sid0, sid1)
    sg = (sg0, sg1)

    def fire_idx(j, b):
        pltpu.async_copy(src_hbm.at[pl.ds(ebase + j * K, K)], sidx[b], sis[b])
        pltpu.async_copy(dst_hbm.at[pl.ds(ebase + j * K, K)], didx[b], sid[b])

    def wait_idx(b):
        pltpu.make_async_copy(src_hbm.at[pl.ds(0, K)], sidx[b], sis[b]).wait()
        pltpu.make_async_copy(dst_hbm.at[pl.ds(0, K)], didx[b], sid[b]).wait()

    def fire_gat(b):
        pltpu.async_copy(g_hbm.at[sidx[b]], bufs[b], sg[b])

    def wait_gat(b):
        pltpu.make_async_copy(g_hbm.at[sidx[b]], bufs[b], sg[b]).wait()

    def scat(b):
        pltpu.sync_copy(bufs[b], acc_sh.at[didx[b]], add=True)

    fire_idx(0, 0)
    wait_idx(0)
    fire_idx(1, 1)
    fire_gat(0)

    def body(m, carry):
        for b in range(2):
            wait_idx((b + 1) % 2)
            fire_gat((b + 1) % 2)
            wait_gat(b)
            scat(b)
            fire_idx(2 * m + b + 2, b)
        return carry

    lax.fori_loop(0, (CH - 2) // 2, body, 0)
    wait_idx(1)
    fire_gat(1)
    wait_gat(0)
    scat(0)
    wait_gat(1)
    scat(1)

    plsc.subcore_barrier()
    pltpu.sync_copy(acc_sh.at[pl.ds(base, RPT)], out_hbm.at[c, pl.ds(base, RPT)])


def _tc_g(x_ref, w_ref, hist_ref, g_ref):
    h = jnp.dot(x_ref[...], w_ref[...], preferred_element_type=jnp.float32)
    deg = jnp.sum(hist_ref[...], axis=1) + 1.0
    dis = lax.rsqrt(deg)[:, None]
    g_ref[...] = h * dis


def _tc_comb(p_ref, g_ref, hist_ref, b_ref, t_ref, s_ref, q_ref):
    deg = jnp.sum(hist_ref[...], axis=1) + 1.0
    dis = lax.rsqrt(deg)[:, None]
    t = dis * (p_ref[0] + p_ref[1] + g_ref[...]) + b_ref[...]
    t_ref[...] = t
    s_ref[...] = jnp.sum(t, axis=0).reshape(1, 1, D)
    q_ref[...] = jnp.sum(t * t, axis=0).reshape(1, 1, D)


def _tc_bn(t_ref, s_ref, q_ref, gam_ref, bet_ref, o_ref):
    inv_n = 1.0 / N
    mean = jnp.sum(s_ref[...], axis=0) * inv_n
    var = jnp.sum(q_ref[...], axis=0) * inv_n - mean * mean
    y = (t_ref[...] - mean) * lax.rsqrt(var + 1e-5) * gam_ref[...] + bet_ref[...]
    o_ref[...] = jnp.maximum(y, 0.0)


_mesh = plsc.VectorSubcoreMesh(core_axis_name="c", subcore_axis_name="s")

_deg_kernel = functools.partial(
    pl.kernel,
    out_type=jax.ShapeDtypeStruct((NW, NROW), jnp.float32),
    mesh=_mesh,
    scratch_types=[
        pltpu.VMEM((EPW,), jnp.int32),
        pltpu.VMEM((NROW + 16,), jnp.float32),
        pltpu.VMEM((NROW + 16,), jnp.float32),
    ],
)(_deg_body)

_IDX = pltpu.VMEM((K,), jnp.int32)
_SEM = pltpu.SemaphoreType.DMA

_main_kernel = functools.partial(
    pl.kernel,
    out_type=jax.ShapeDtypeStruct((2, NROW, D), jnp.float32),
    mesh=_mesh,
    scratch_types=[
        _IDX, _IDX, _IDX, _IDX,
        pltpu.VMEM((K, D), jnp.float32),
        pltpu.VMEM((K, D), jnp.float32),
        _SEM, _SEM, _SEM, _SEM, _SEM, _SEM,
        pltpu.VMEM_SHARED((NROW, D), jnp.float32),
    ],
)(_main_body)


def kernel_debug(x, edge_index, W, b, gamma, beta, use_sc_deg, use_sc_main):
    import jax.ops
    src0, dst0 = edge_index[0], edge_index[1]
    srcp = jnp.concatenate([src0, jnp.zeros((E_PAD - E,), jnp.int32)])
    dstp = jnp.concatenate([dst0, jnp.full((E_PAD - E,), N, jnp.int32)])
    zeros16 = jnp.zeros((NROW, 16), jnp.float32)
    ones16 = jnp.ones((K, 16), jnp.float32)
    zeros128 = jnp.zeros((NROW, D), jnp.float32)

    if use_sc_deg:
        degp = _deg_kernel(dstp)
        deg = jnp.sum(degp[:, :N], axis=0) + 1.0
    else:
        ones = jnp.ones((E,), jnp.float32)
        deg = jax.ops.segment_sum(ones, dst0, num_segments=N) + 1.0
    dis = jax.lax.rsqrt(deg)
    g = (x @ W) * dis[:, None]
    if use_sc_main:
        parts = _main_kernel(g, srcp, dstp, zeros128)
        S = parts[0, :N] + parts[1, :N]
    else:
        S = jax.ops.segment_sum(g[src0], dst0, num_segments=N)
    out = dis[:, None] * (S + g) + b
    mean = jnp.mean(out, axis=0)
    var = jnp.var(out, axis=0)
    out = (out - mean) * jax.lax.rsqrt(var + 1e-5) * gamma + beta
    return jnp.maximum(out, 0.0)


def kernel(x, edge_index, W, b, gamma, beta):
    src = jnp.concatenate([edge_index[0], jnp.zeros((E_PAD - E,), jnp.int32)])
    dst = jnp.concatenate([edge_index[1], jnp.full((E_PAD - E,), N, jnp.int32)])

    zeros128 = jnp.zeros((NROW, D), jnp.float32)
    b2 = b.reshape(1, D)
    gamma2 = gamma.reshape(1, D)
    beta2 = beta.reshape(1, D)

    degp = _deg_kernel(dst).T

    g = pl.pallas_call(
        _tc_g,
        grid=(GRID,),
        in_specs=[
            pl.BlockSpec((BLK, D), lambda i: (i, 0)),
            pl.BlockSpec((D, D), lambda i: (0, 0)),
            pl.BlockSpec((BLK, NW), lambda i: (i, 0)),
        ],
        out_specs=pl.BlockSpec((BLK, D), lambda i: (i, 0)),
        out_shape=jax.ShapeDtypeStruct((N, D), jnp.float32),
    )(x, W, degp)

    parts = _main_kernel(g, src, dst, zeros128)

    t, sums, sq = pl.pallas_call(
        _tc_comb,
        grid=(GRID,),
        in_specs=[
            pl.BlockSpec((2, BLK, D), lambda i: (0, i, 0)),
            pl.BlockSpec((BLK, D), lambda i: (i, 0)),
            pl.BlockSpec((BLK, NW), lambda i: (i, 0)),
            pl.BlockSpec((1, D), lambda i: (0, 0)),
        ],
        out_specs=[
            pl.BlockSpec((BLK, D), lambda i: (i, 0)),
            pl.BlockSpec((1, 1, D), lambda i: (i, 0, 0)),
            pl.BlockSpec((1, 1, D), lambda i: (i, 0, 0)),
        ],
        out_shape=[
            jax.ShapeDtypeStruct((N, D), jnp.float32),
            jax.ShapeDtypeStruct((GRID, 1, D), jnp.float32),
            jax.ShapeDtypeStruct((GRID, 1, D), jnp.float32),
        ],
    )(parts, g, degp, b2)

    out = pl.pallas_call(
        _tc_bn,
        grid=(GRID,),
        in_specs=[
            pl.BlockSpec((BLK, D), lambda i: (i, 0)),
            pl.BlockSpec((GRID, 1, D), lambda i: (0, 0, 0)),
            pl.BlockSpec((GRID, 1, D), lambda i: (0, 0, 0)),
            pl.BlockSpec((1, D), lambda i: (0, 0)),
            pl.BlockSpec((1, D), lambda i: (0, 0)),
        ],
        out_specs=pl.BlockSpec((BLK, D), lambda i: (i, 0)),
        out_shape=jax.ShapeDtypeStruct((N, D), jnp.float32),
    )(t, sums, sq, gamma2, beta2)

    return out
